# Initial kernel scaffold; baseline (speedup 1.0000x reference)
import jax
import jax.numpy as jnp
from jax.experimental import pallas as pl


def kernel(pred, target):
    raise NotImplementedError("write your pallas kernel here")


# speedup vs baseline: 2.5664x; 2.5664x over previous
import functools

import jax
import jax.numpy as jnp
import numpy as np
from jax.experimental import pallas as pl
from jax.experimental.pallas import tpu as pltpu

_N_CORES = 2
_CHUNK = 8
_LANE_PAD = 256


def _yolo_kernel(p_ref, t_ref, out_ref, acc_ref, *,
                 lambda_coord, lambda_noobj, n_lane, unroll):
    j = pl.program_id(1)

    @pl.when(j == 0)
    def _():
        acc_ref[...] = jnp.zeros_like(acc_ref)

    bb = p_ref.shape[0]
    n_chunks = bb // _CHUNK

    lane = jax.lax.broadcasted_iota(jnp.int32, (_CHUNK, _LANE_PAD), 1)
    m0 = (lane % 5 == 0) & (lane < n_lane)
    zpad = jnp.zeros((_CHUNK, _LANE_PAD - n_lane), jnp.float32)

    def rl(x, k):
        return pltpu.roll(x, _LANE_PAD - k, 1)

    def chunk(c, carry):
        a_loc, a_co, a_cn = carry
        r0 = pl.multiple_of(c * _CHUNK, _CHUNK)
        p = jnp.concatenate([p_ref[pl.ds(r0, _CHUNK), :], zpad], axis=1)
        t = jnp.concatenate([t_ref[pl.ds(r0, _CHUNK), :], zpad], axis=1)

        d = p - t
        sq = d * d
        loc = sq + rl(sq, 1) + rl(sq, 2) + rl(sq, 3)

        pwh = rl(p, 2) * 0.5
        twh = rl(t, 2) * 0.5
        lo = jnp.maximum(p - pwh, t - twh)
        hi = jnp.minimum(p + pwh, t + twh)
        o = jnp.maximum(hi - lo, 0.0)
        inter = o * rl(o, 1)

        area = p * rl(p, 1) + t * rl(t, 1)
        union = rl(area, 2) - inter
        iou = inter / (union + 1e-6)

        pc = rl(p, 4)
        tc = rl(t, 4)
        dob = iou - pc
        dno = pc - tc
        obj = m0 & (tc > 0.0)
        nob = m0 & (tc == 0.0)

        return (a_loc + jnp.where(obj, loc, 0.0),
                a_co + jnp.where(obj, dob * dob, 0.0),
                a_cn + jnp.where(nob, dno * dno, 0.0))

    z = jnp.zeros((_CHUNK, _LANE_PAD), jnp.float32)
    a_loc, a_co, a_cn = jax.lax.fori_loop(0, n_chunks, chunk, (z, z, z),
                                          unroll=unroll)
    acc_ref[0] += a_loc
    acc_ref[1] += a_co
    acc_ref[2] += a_cn

    @pl.when(j == pl.num_programs(1) - 1)
    def _():
        loc = lambda_coord * jnp.sum(acc_ref[0])
        co = jnp.sum(acc_ref[1])
        cn = lambda_noobj * jnp.sum(acc_ref[2])
        tot = loc + co + cn
        ol = jax.lax.broadcasted_iota(jnp.int32, (8, 128), 1)
        out_ref[0] = jnp.where(ol == 0, tot,
                     jnp.where(ol == 1, loc,
                     jnp.where(ol == 2, co,
                     jnp.where(ol == 3, cn, 0.0))))


@functools.partial(jax.jit, static_argnames=("lambda_coord", "lambda_noobj"))
def _yolo_v1_loss(pred, target, lambda_coord=5.0, lambda_noobj=0.5):
    assert pred.shape == target.shape and pred.shape[-1] == 5
    b = pred.shape[0]
    n_lane = int(np.prod(pred.shape[1:]))
    p2 = pred.reshape(b, n_lane)
    t2 = target.reshape(b, n_lane)

    half = b // _N_CORES
    bb = 0
    for cand in range(min(half, 2048), 7, -1):
        if cand % 8 == 0 and half % cand == 0:
            bb = cand
            break
    assert bb, f"no row blocking for batch {b}"
    nb = half // bb
    unroll = 4 if (bb // _CHUNK) % 4 == 0 else 1

    itemsize = np.dtype(pred.dtype).itemsize
    cost = pl.CostEstimate(
        flops=64 * b * n_lane,
        transcendentals=0,
        bytes_accessed=2 * b * n_lane * itemsize + 2 * 8 * 128 * 4)

    body = functools.partial(
        _yolo_kernel,
        lambda_coord=float(lambda_coord),
        lambda_noobj=float(lambda_noobj),
        n_lane=n_lane,
        unroll=unroll)

    out = pl.pallas_call(
        body,
        out_shape=jax.ShapeDtypeStruct((_N_CORES, 8, 128), jnp.float32),
        grid=(_N_CORES, nb),
        in_specs=[
            pl.BlockSpec((bb, n_lane), lambda i, j, nb=nb: (i * nb + j, 0)),
            pl.BlockSpec((bb, n_lane), lambda i, j, nb=nb: (i * nb + j, 0)),
        ],
        out_specs=pl.BlockSpec((1, 8, 128), lambda i, j: (i, 0, 0)),
        scratch_shapes=[pltpu.VMEM((3, _CHUNK, _LANE_PAD), jnp.float32)],
        compiler_params=pltpu.CompilerParams(
            dimension_semantics=("parallel", "arbitrary"),
            vmem_limit_bytes=32 * 1024 * 1024),
        cost_estimate=cost,
    )(p2, t2)

    totals = jnp.sum(out[:, 0, :4], axis=0)
    return totals[0], totals[1], totals[2], totals[3]


def kernel(pred, target):
    return _yolo_v1_loss(pred, target)


# speedup vs baseline: 4.7509x; 1.8512x over previous
import functools

import jax
import jax.numpy as jnp
import numpy as np
from jax.experimental import pallas as pl
from jax.experimental.pallas import tpu as pltpu

_N_CORES = 2
_CHUNK = 8
_LANE_PAD = 256


def _yolo_kernel(p_ref, t_ref, out_ref, acc_ref, *,
                 lambda_coord, lambda_noobj, n_lane, unroll):
    j = pl.program_id(1)

    @pl.when(j == 0)
    def _():
        acc_ref[...] = jnp.zeros_like(acc_ref)

    bb = p_ref.shape[0]
    n_chunks = bb // _CHUNK

    lane = jax.lax.broadcasted_iota(jnp.int32, (_CHUNK, _LANE_PAD), 1)
    m0 = (lane % 5 == 0) & (lane < n_lane)

    def rl(x, k):
        return pltpu.roll(x, _LANE_PAD - k, 1)

    def chunk(c, carry):
        a_loc, a_co, a_cn = carry
        r0 = pl.multiple_of(c * _CHUNK, _CHUNK)
        p = p_ref[pl.ds(r0, _CHUNK), :]
        t = t_ref[pl.ds(r0, _CHUNK), :]

        d = p - t
        sq = d * d
        loc = sq + rl(sq, 1) + rl(sq, 2) + rl(sq, 3)

        pwh = rl(p, 2) * 0.5
        twh = rl(t, 2) * 0.5
        lo = jnp.maximum(p - pwh, t - twh)
        hi = jnp.minimum(p + pwh, t + twh)
        o = jnp.maximum(hi - lo, 0.0)
        inter = o * rl(o, 1)

        area = p * rl(p, 1) + t * rl(t, 1)
        union = rl(area, 2) - inter
        iou = inter / (union + 1e-6)

        pc = rl(p, 4)
        tc = rl(t, 4)
        dob = iou - pc
        dno = pc - tc
        obj = m0 & (tc > 0.0)
        nob = m0 & (tc == 0.0)

        return (a_loc + jnp.where(obj, loc, 0.0),
                a_co + jnp.where(obj, dob * dob, 0.0),
                a_cn + jnp.where(nob, dno * dno, 0.0))

    z = jnp.zeros((_CHUNK, _LANE_PAD), jnp.float32)
    a_loc, a_co, a_cn = jax.lax.fori_loop(0, n_chunks, chunk, (z, z, z),
                                          unroll=unroll)
    acc_ref[0] += a_loc
    acc_ref[1] += a_co
    acc_ref[2] += a_cn

    @pl.when(j == pl.num_programs(1) - 1)
    def _():
        loc = lambda_coord * jnp.sum(acc_ref[0])
        co = jnp.sum(acc_ref[1])
        cn = lambda_noobj * jnp.sum(acc_ref[2])
        tot = loc + co + cn
        ol = jax.lax.broadcasted_iota(jnp.int32, (8, 128), 1)
        out_ref[0] = jnp.where(ol == 0, tot,
                     jnp.where(ol == 1, loc,
                     jnp.where(ol == 2, co,
                     jnp.where(ol == 3, cn, 0.0))))


@functools.partial(jax.jit, static_argnames=("lambda_coord", "lambda_noobj"))
def _yolo_v1_loss(pred, target, lambda_coord=5.0, lambda_noobj=0.5):
    assert pred.shape == target.shape and pred.shape[-1] == 5
    b = pred.shape[0]
    n_lane = int(np.prod(pred.shape[1:]))
    p2 = pred.reshape(b, n_lane)
    t2 = target.reshape(b, n_lane)

    half = b // _N_CORES
    bb = 0
    for cand in range(min(half, 2048), 7, -1):
        if cand % 8 == 0 and half % cand == 0:
            bb = cand
            break
    assert bb, f"no row blocking for batch {b}"
    nb = half // bb
    n_chunks = bb // _CHUNK
    unroll = next((u for u in (8, 5, 4, 2) if n_chunks % u == 0), 1)

    itemsize = np.dtype(pred.dtype).itemsize
    cost = pl.CostEstimate(
        flops=64 * b * n_lane,
        transcendentals=0,
        bytes_accessed=2 * b * n_lane * itemsize + 2 * 8 * 128 * 4)

    body = functools.partial(
        _yolo_kernel,
        lambda_coord=float(lambda_coord),
        lambda_noobj=float(lambda_noobj),
        n_lane=n_lane,
        unroll=unroll)

    out = pl.pallas_call(
        body,
        out_shape=jax.ShapeDtypeStruct((_N_CORES, 8, 128), jnp.float32),
        grid=(_N_CORES, nb),
        in_specs=[
            pl.BlockSpec((bb, _LANE_PAD), lambda i, j, nb=nb: (i * nb + j, 0)),
            pl.BlockSpec((bb, _LANE_PAD), lambda i, j, nb=nb: (i * nb + j, 0)),
        ],
        out_specs=pl.BlockSpec((1, 8, 128), lambda i, j: (i, 0, 0)),
        scratch_shapes=[pltpu.VMEM((3, _CHUNK, _LANE_PAD), jnp.float32)],
        compiler_params=pltpu.CompilerParams(
            dimension_semantics=("parallel", "arbitrary"),
            vmem_limit_bytes=32 * 1024 * 1024),
        cost_estimate=cost,
    )(p2, t2)

    totals = jnp.sum(out[:, 0, :4], axis=0)
    return totals[0], totals[1], totals[2], totals[3]


def kernel(pred, target):
    return _yolo_v1_loss(pred, target)


# speedup vs baseline: 67.6771x; 14.2451x over previous
import functools

import jax
import jax.numpy as jnp
import numpy as np
from jax.experimental import pallas as pl
from jax.experimental.pallas import tpu as pltpu

_N_CORES = 2
_NBL = 1024


def _yolo_kernel(p_ref, t_ref, out_ref, acc_ref, *,
                 lambda_coord, lambda_noobj, n_b, unroll):
    j = pl.program_id(1)

    @pl.when(j == 0)
    def _():
        acc_ref[...] = jnp.zeros_like(acc_ref)

    s1n, cn, s2n, nbl = p_ref.shape
    base = (pl.program_id(0) * pl.num_programs(1) + j) * nbl
    iota = jax.lax.broadcasted_iota(jnp.int32, (s2n, 128), 1)

    def lt_body(lt, carry):
        a_loc, a_co, a_cn = carry
        l0 = pl.multiple_of(lt * 128, 128)
        valid = iota < n_b - (base + lt * 128)
        for s1 in range(s1n):
            px, py, pw, ph, pc = (
                p_ref[s1, c, :, pl.ds(l0, 128)] for c in range(cn))
            tx, ty, tw, th, tc = (
                t_ref[s1, c, :, pl.ds(l0, 128)] for c in range(cn))

            dx = px - tx
            dy = py - ty
            dw = pw - tw
            dh = ph - th
            loc = dx * dx + dy * dy + dw * dw + dh * dh

            phw = pw * 0.5
            phh = ph * 0.5
            thw = tw * 0.5
            thh = th * 0.5
            iw = jnp.maximum(
                jnp.minimum(px + phw, tx + thw)
                - jnp.maximum(px - phw, tx - thw), 0.0)
            ih = jnp.maximum(
                jnp.minimum(py + phh, ty + thh)
                - jnp.maximum(py - phh, ty - thh), 0.0)
            inter = iw * ih
            union = pw * ph + tw * th - inter
            iou = inter / (union + 1e-6)

            dob = iou - pc
            dno = pc - tc
            obj = valid & (tc > 0.0)
            nob = valid & (tc == 0.0)

            a_loc = a_loc + jnp.where(obj, loc, 0.0)
            a_co = a_co + jnp.where(obj, dob * dob, 0.0)
            a_cn = a_cn + jnp.where(nob, dno * dno, 0.0)
        return a_loc, a_co, a_cn

    z = jnp.zeros((s2n, 128), jnp.float32)
    a_loc, a_co, a_cn = jax.lax.fori_loop(0, nbl // 128, lt_body, (z, z, z),
                                          unroll=unroll)
    acc_ref[0] += a_loc
    acc_ref[1] += a_co
    acc_ref[2] += a_cn

    @pl.when(j == pl.num_programs(1) - 1)
    def _():
        loc = lambda_coord * jnp.sum(acc_ref[0])
        co = jnp.sum(acc_ref[1])
        cn = lambda_noobj * jnp.sum(acc_ref[2])
        tot = loc + co + cn
        ol = jax.lax.broadcasted_iota(jnp.int32, (8, 128), 1)
        out_ref[0] = jnp.where(ol == 0, tot,
                     jnp.where(ol == 1, loc,
                     jnp.where(ol == 2, co,
                     jnp.where(ol == 3, cn, 0.0))))


@functools.partial(jax.jit, static_argnames=("lambda_coord", "lambda_noobj"))
def _yolo_v1_loss(pred, target, lambda_coord=5.0, lambda_noobj=0.5):
    assert pred.shape == target.shape and pred.shape[-1] == 5
    b, s1, s2, c = pred.shape

    pt = jnp.transpose(pred, (1, 3, 2, 0))
    tt = jnp.transpose(target, (1, 3, 2, 0))

    nbl = _NBL
    while nbl > 128:
        nb = -(-b // (_N_CORES * nbl))
        if (_N_CORES * nb - 1) * nbl < b:
            break
        nbl //= 2
    nb = -(-b // (_N_CORES * nbl))
    assert (_N_CORES * nb - 1) * nbl < b, f"batch {b} too small to block"
    n_lt = nbl // 128
    unroll = next((u for u in (4, 2) if n_lt % u == 0), 1)

    itemsize = np.dtype(pred.dtype).itemsize
    cost = pl.CostEstimate(
        flops=64 * b * s1 * s2 * c,
        transcendentals=0,
        bytes_accessed=2 * b * s1 * s2 * c * itemsize + 2 * 8 * 128 * 4)

    body = functools.partial(
        _yolo_kernel,
        lambda_coord=float(lambda_coord),
        lambda_noobj=float(lambda_noobj),
        n_b=b, unroll=unroll)

    out = pl.pallas_call(
        body,
        out_shape=jax.ShapeDtypeStruct((_N_CORES, 8, 128), jnp.float32),
        grid=(_N_CORES, nb),
        in_specs=[
            pl.BlockSpec((s1, c, s2, nbl),
                         lambda i, j, nb=nb: (0, 0, 0, i * nb + j)),
            pl.BlockSpec((s1, c, s2, nbl),
                         lambda i, j, nb=nb: (0, 0, 0, i * nb + j)),
        ],
        out_specs=pl.BlockSpec((1, 8, 128), lambda i, j: (i, 0, 0)),
        scratch_shapes=[pltpu.VMEM((3, s2, 128), jnp.float32)],
        compiler_params=pltpu.CompilerParams(
            dimension_semantics=("parallel", "arbitrary"),
            vmem_limit_bytes=32 * 1024 * 1024),
        cost_estimate=cost,
    )(pt, tt)

    totals = jnp.sum(out[:, 0, :4], axis=0)
    return totals[0], totals[1], totals[2], totals[3]


def kernel(pred, target):
    return _yolo_v1_loss(pred, target)


# speedup vs baseline: 88.3662x; 1.3057x over previous
import functools

import jax
import jax.numpy as jnp
import numpy as np
from jax.experimental import pallas as pl
from jax.experimental.pallas import tpu as pltpu


def _yolo_kernel(p_ref, t_ref, out_ref, *, lambda_coord, lambda_noobj,
                 n_b, unroll):
    cn = p_ref.shape[1]
    s2n = p_ref.shape[2]
    n_full = n_b // 128
    n_tail = n_b - n_full * 128

    def cell_terms(l0, width):
        px, py, pw, ph, pc = (
            p_ref[0, c, :, pl.ds(l0, width)] for c in range(cn))
        tx, ty, tw, th, tc = (
            t_ref[0, c, :, pl.ds(l0, width)] for c in range(cn))

        dx = px - tx
        dy = py - ty
        dw = pw - tw
        dh = ph - th
        loc = dx * dx + dy * dy + dw * dw + dh * dh

        phw = pw * 0.5
        phh = ph * 0.5
        thw = tw * 0.5
        thh = th * 0.5
        iw = jnp.maximum(
            jnp.minimum(px + phw, tx + thw)
            - jnp.maximum(px - phw, tx - thw), 0.0)
        ih = jnp.maximum(
            jnp.minimum(py + phh, ty + thh)
            - jnp.maximum(py - phh, ty - thh), 0.0)
        inter = iw * ih
        union = pw * ph + tw * th - inter
        iou = inter / (union + 1e-6)

        dob = iou - pc
        dno = pc - tc
        obj = tc > 0.0
        nob = tc == 0.0
        return (jnp.where(obj, loc, 0.0),
                jnp.where(obj, dob * dob, 0.0),
                jnp.where(nob, dno * dno, 0.0))

    def lt_body(lt, carry):
        a_loc, a_co, a_cn = carry
        l0 = pl.multiple_of(lt * 128, 128)
        c_loc, c_co, c_cn = cell_terms(l0, 128)
        return a_loc + c_loc, a_co + c_co, a_cn + c_cn

    z = jnp.zeros((s2n, 128), jnp.float32)
    a_loc, a_co, a_cn = jax.lax.fori_loop(0, n_full, lt_body, (z, z, z),
                                          unroll=unroll)
    loc = jnp.sum(a_loc)
    co = jnp.sum(a_co)
    cn_ = jnp.sum(a_cn)
    if n_tail:
        t_loc, t_co, t_cn = cell_terms(n_full * 128, n_tail)
        loc = loc + jnp.sum(t_loc)
        co = co + jnp.sum(t_co)
        cn_ = cn_ + jnp.sum(t_cn)

    loc = lambda_coord * loc
    cn_ = lambda_noobj * cn_
    tot = loc + co + cn_
    ol = jax.lax.broadcasted_iota(jnp.int32, (8, 128), 1)
    out_ref[0] = jnp.where(ol == 0, tot,
                 jnp.where(ol == 1, loc,
                 jnp.where(ol == 2, co,
                 jnp.where(ol == 3, cn_, 0.0))))


@functools.partial(jax.jit, static_argnames=("lambda_coord", "lambda_noobj"))
def _yolo_v1_loss(pred, target, lambda_coord=5.0, lambda_noobj=0.5):
    assert pred.shape == target.shape and pred.shape[-1] == 5
    b, s1, s2, c = pred.shape

    pt = jnp.transpose(pred, (1, 3, 2, 0))
    tt = jnp.transpose(target, (1, 3, 2, 0))

    itemsize = np.dtype(pred.dtype).itemsize
    cost = pl.CostEstimate(
        flops=64 * b * s1 * s2 * c,
        transcendentals=0,
        bytes_accessed=2 * b * s1 * s2 * c * itemsize + s1 * 8 * 128 * 4)

    body = functools.partial(
        _yolo_kernel,
        lambda_coord=float(lambda_coord),
        lambda_noobj=float(lambda_noobj),
        n_b=b, unroll=4)

    out = pl.pallas_call(
        body,
        out_shape=jax.ShapeDtypeStruct((s1, 8, 128), jnp.float32),
        grid=(s1,),
        in_specs=[
            pl.BlockSpec((1, c, s2, b), lambda k: (k, 0, 0, 0)),
            pl.BlockSpec((1, c, s2, b), lambda k: (k, 0, 0, 0)),
        ],
        out_specs=pl.BlockSpec((1, 8, 128), lambda k: (k, 0, 0)),
        compiler_params=pltpu.CompilerParams(
            dimension_semantics=("parallel",),
            vmem_limit_bytes=48 * 1024 * 1024),
        cost_estimate=cost,
    )(pt, tt)

    totals = jnp.sum(out[:, 0, :4], axis=0)
    return totals[0], totals[1], totals[2], totals[3]


def kernel(pred, target):
    return _yolo_v1_loss(pred, target)
